# baseline (device time: 35456 ns/iter reference)
import jax
import jax.numpy as jnp
from jax import lax
from jax.experimental import pallas as pl
from jax.experimental.pallas import tpu as pltpu

N_DEV = 4
SQ = 256
D = 1024
SKV = 4096
DH = 128
HQ_SHARD = 8
KV_SHARD = 2
QC = D // 4
SCALE = 0.08838834764831843

CHUNKS = ((0, 96), (96, 96), (192, 64))
NC = len(CHUNKS)
MAXR = max(r for _, r in CHUNKS)


def kernel(x, Wq, Wo, K_ext, V_ext):
    my_pos = lax.axis_index("i").astype(jnp.int32)

    def body(pos_ref, x_ref, wq_ref, wo_ref, k_any, v_any, out_ref,
             kv_ref, vv_ref, attn_ref, acc_ref, stage_ref, fin_ref,
             kv_sems, send_sems, recv_sems):
        my = pos_ref[0]
        peers = [jnp.bitwise_xor(my, j + 1) for j in range(3)]

        kv_copies = []
        for g in range(KV_SHARD):
            hd = 2 * my + g
            for j, (src, dst) in enumerate(((k_any, kv_ref), (v_any, vv_ref))):
                cp = pltpu.make_async_copy(
                    src.at[0, :, hd, :], dst.at[g], kv_sems.at[2 * g + j])
                cp.start()
                kv_copies.append(cp)

        barrier = pltpu.get_barrier_semaphore()
        for nbr in peers:
            pl.semaphore_signal(barrier, inc=1, device_id=(nbr,),
                                device_id_type=pl.DeviceIdType.MESH)
        pl.semaphore_wait(barrier, 3)

        q = jnp.dot(x_ref[0], wq_ref[:, :],
                    preferred_element_type=jnp.float32)
        qs = q * SCALE

        kv_waited = [False, False]

        def attn_head(c, h):
            g = h // 4
            if not kv_waited[g]:
                kv_copies[2 * g].wait()
                kv_copies[2 * g + 1].wait()
                kv_waited[g] = True
            base, rows = CHUNKS[c]
            q_h = qs[base:base + rows, h * DH:(h + 1) * DH]
            k_h = kv_ref[g]
            v_h = vv_ref[g]
            s = lax.dot_general(q_h, k_h, (((1,), (1,)), ((), ())),
                                preferred_element_type=jnp.float32)
            p = jnp.exp(s)
            l = jnp.sum(p, axis=1, keepdims=True)
            o = jnp.dot(p, v_h, preferred_element_type=jnp.float32) / l
            attn_ref[base:base + rows, h * DH:(h + 1) * DH] = o

        def finish_chunk(c):
            base, rows = CHUNKS[c]
            partial = jnp.dot(attn_ref[base:base + rows, :], wo_ref[:, :],
                              preferred_element_type=jnp.float32)
            for qq in range(N_DEV):
                acc_ref[c, qq, 0:rows] = partial[:, qq * QC:(qq + 1) * QC]

        def phase1_start(c):
            rows = CHUNKS[c][1]
            ds = []
            for j, p in enumerate(peers):
                d = pltpu.make_async_remote_copy(
                    src_ref=acc_ref.at[c, pl.ds(p, 1), pl.ds(0, rows)],
                    dst_ref=stage_ref.at[c, pl.ds(j, 1), pl.ds(0, rows)],
                    send_sem=send_sems.at[c, 0, j],
                    recv_sem=recv_sems.at[c, 0, j],
                    device_id=(p,),
                    device_id_type=pl.DeviceIdType.MESH,
                )
                d.start()
                ds.append(d)
            return ds

        def phase1_reduce(c):
            rows = CHUNKS[c][1]
            for j in range(3):
                pltpu.make_async_remote_copy(
                    src_ref=acc_ref.at[c, pl.ds(peers[j], 1), pl.ds(0, rows)],
                    dst_ref=stage_ref.at[c, pl.ds(j, 1), pl.ds(0, rows)],
                    send_sem=send_sems.at[c, 0, j],
                    recv_sem=recv_sems.at[c, 0, j],
                    device_id=(peers[j],),
                    device_id_type=pl.DeviceIdType.MESH,
                ).wait_recv()
            red = (acc_ref[c, pl.ds(my, 1), 0:rows]
                   + (stage_ref[c, 0:1, 0:rows] + stage_ref[c, 1:2, 0:rows])
                   + stage_ref[c, 2:3, 0:rows])
            fin_ref[c, pl.ds(my, 1), 0:rows] = red

        def phase2_start(c):
            rows = CHUNKS[c][1]
            ds = []
            for j, p in enumerate(peers):
                d = pltpu.make_async_remote_copy(
                    src_ref=fin_ref.at[c, pl.ds(my, 1), pl.ds(0, rows)],
                    dst_ref=fin_ref.at[c, pl.ds(my, 1), pl.ds(0, rows)],
                    send_sem=send_sems.at[c, 1, j],
                    recv_sem=recv_sems.at[c, 1, j],
                    device_id=(p,),
                    device_id_type=pl.DeviceIdType.MESH,
                )
                d.start()
                ds.append(d)
            return ds

        def phase2_wait(c):
            rows = CHUNKS[c][1]
            for j in range(3):
                pltpu.make_async_remote_copy(
                    src_ref=fin_ref.at[c, pl.ds(my, 1), pl.ds(0, rows)],
                    dst_ref=fin_ref.at[c, pl.ds(peers[j], 1), pl.ds(0, rows)],
                    send_sem=send_sems.at[c, 1, j],
                    recv_sem=recv_sems.at[c, 1, j],
                    device_id=(peers[j],),
                    device_id_type=pl.DeviceIdType.MESH,
                ).wait_recv()

        sends = []
        for h in range(HQ_SHARD):
            attn_head(0, h)
        finish_chunk(0)
        sends += phase1_start(0)
        for c in range(1, NC):
            for h in range(6):
                attn_head(c, h)
            phase1_reduce(c - 1)
            sends += phase2_start(c - 1)
            for h in range(6, HQ_SHARD):
                attn_head(c, h)
            finish_chunk(c)
            sends += phase1_start(c)
        phase1_reduce(NC - 1)
        sends += phase2_start(NC - 1)
        for c in range(NC):
            phase2_wait(c)
        for d in sends:
            d.wait_send()

        for c, (base, rows) in enumerate(CHUNKS):
            for qq in range(N_DEV):
                out_ref[0, base:base + rows,
                        qq * QC:(qq + 1) * QC] = fin_ref[c, qq, 0:rows]

    grid_spec = pltpu.PrefetchScalarGridSpec(
        num_scalar_prefetch=1,
        grid=(1,),
        in_specs=[
            pl.BlockSpec((1, SQ, D), lambda i, m: (0, 0, 0)),
            pl.BlockSpec((D, D), lambda i, m: (0, 0)),
            pl.BlockSpec((D, D), lambda i, m: (0, 0)),
            pl.BlockSpec(memory_space=pl.ANY),
            pl.BlockSpec(memory_space=pl.ANY),
        ],
        out_specs=pl.BlockSpec((1, SQ, D), lambda i, m: (0, 0, 0)),
        scratch_shapes=[
            pltpu.VMEM((KV_SHARD, SKV, DH), jnp.float32),
            pltpu.VMEM((KV_SHARD, SKV, DH), jnp.float32),
            pltpu.VMEM((SQ, D), jnp.float32),
            pltpu.VMEM((NC, N_DEV, MAXR, QC), jnp.float32),
            pltpu.VMEM((NC, 3, MAXR, QC), jnp.float32),
            pltpu.VMEM((NC, N_DEV, MAXR, QC), jnp.float32),
            pltpu.SemaphoreType.DMA((4,)),
            pltpu.SemaphoreType.DMA((NC, 2, 3)),
            pltpu.SemaphoreType.DMA((NC, 2, 3)),
        ],
    )
    return pl.pallas_call(
        body,
        grid_spec=grid_spec,
        out_shape=jax.ShapeDtypeStruct((1, SQ, D), jnp.float32),
        compiler_params=pltpu.CompilerParams(collective_id=0),
    )(my_pos.reshape(1), x, Wq, Wo, K_ext, V_ext)


# device time: 34751 ns/iter; 1.0203x vs baseline; 1.0203x over previous
import jax
import jax.numpy as jnp
from jax import lax
from jax.experimental import pallas as pl
from jax.experimental.pallas import tpu as pltpu

N_DEV = 4
SQ = 256
D = 1024
SKV = 4096
DH = 128
HQ_SHARD = 8
KV_SHARD = 2
QC = D // 4
SCALE = 0.08838834764831843

CHUNKS = ((0, 144), (144, 112))
NC = len(CHUNKS)
MAXR = max(r for _, r in CHUNKS)


def kernel(x, Wq, Wo, K_ext, V_ext):
    my_pos = lax.axis_index("i").astype(jnp.int32)

    def body(pos_ref, x_ref, wq_ref, wo_ref, k_any, v_any, out_ref,
             kv_ref, vv_ref, attn_ref, acc_ref, stage_ref, fin_ref,
             kv_sems, send_sems, recv_sems):
        my = pos_ref[0]
        peers = [jnp.bitwise_xor(my, j + 1) for j in range(3)]

        kv_copies = []
        for g in range(KV_SHARD):
            hd = 2 * my + g
            for j, (src, dst) in enumerate(((k_any, kv_ref), (v_any, vv_ref))):
                cp = pltpu.make_async_copy(
                    src.at[0, :, hd, :], dst.at[g], kv_sems.at[2 * g + j])
                cp.start()
                kv_copies.append(cp)

        barrier = pltpu.get_barrier_semaphore()
        for nbr in peers:
            pl.semaphore_signal(barrier, inc=1, device_id=(nbr,),
                                device_id_type=pl.DeviceIdType.MESH)
        pl.semaphore_wait(barrier, 3)

        q = jnp.dot(x_ref[0], wq_ref[:, :],
                    preferred_element_type=jnp.float32)
        qs = q * SCALE

        kv_waited = [False, False]

        def attn_head(c, h):
            g = h // 4
            if not kv_waited[g]:
                kv_copies[2 * g].wait()
                kv_copies[2 * g + 1].wait()
                kv_waited[g] = True
            base, rows = CHUNKS[c]
            q_h = qs[base:base + rows, h * DH:(h + 1) * DH]
            k_h = kv_ref[g]
            v_h = vv_ref[g]
            s = lax.dot_general(q_h, k_h, (((1,), (1,)), ((), ())),
                                preferred_element_type=jnp.float32)
            p = jnp.exp(s)
            l = jnp.sum(p, axis=1, keepdims=True)
            o = jnp.dot(p, v_h, preferred_element_type=jnp.float32) / l
            attn_ref[base:base + rows, h * DH:(h + 1) * DH] = o

        def finish_chunk(c):
            base, rows = CHUNKS[c]
            partial = jnp.dot(attn_ref[base:base + rows, :], wo_ref[:, :],
                              preferred_element_type=jnp.float32)
            for qq in range(N_DEV):
                acc_ref[c, qq, 0:rows] = partial[:, qq * QC:(qq + 1) * QC]

        def phase1_start(c):
            rows = CHUNKS[c][1]
            ds = []
            for j, p in enumerate(peers):
                d = pltpu.make_async_remote_copy(
                    src_ref=acc_ref.at[c, pl.ds(p, 1), pl.ds(0, rows)],
                    dst_ref=stage_ref.at[c, pl.ds(j, 1), pl.ds(0, rows)],
                    send_sem=send_sems.at[c, 0, j],
                    recv_sem=recv_sems.at[c, 0, j],
                    device_id=(p,),
                    device_id_type=pl.DeviceIdType.MESH,
                )
                d.start()
                ds.append(d)
            return ds

        def phase1_reduce(c):
            rows = CHUNKS[c][1]
            for j in range(3):
                pltpu.make_async_remote_copy(
                    src_ref=acc_ref.at[c, pl.ds(peers[j], 1), pl.ds(0, rows)],
                    dst_ref=stage_ref.at[c, pl.ds(j, 1), pl.ds(0, rows)],
                    send_sem=send_sems.at[c, 0, j],
                    recv_sem=recv_sems.at[c, 0, j],
                    device_id=(peers[j],),
                    device_id_type=pl.DeviceIdType.MESH,
                ).wait_recv()
            red = (acc_ref[c, pl.ds(my, 1), 0:rows]
                   + (stage_ref[c, 0:1, 0:rows] + stage_ref[c, 1:2, 0:rows])
                   + stage_ref[c, 2:3, 0:rows])
            fin_ref[c, pl.ds(my, 1), 0:rows] = red

        def phase2_start(c):
            rows = CHUNKS[c][1]
            ds = []
            for j, p in enumerate(peers):
                d = pltpu.make_async_remote_copy(
                    src_ref=fin_ref.at[c, pl.ds(my, 1), pl.ds(0, rows)],
                    dst_ref=fin_ref.at[c, pl.ds(my, 1), pl.ds(0, rows)],
                    send_sem=send_sems.at[c, 1, j],
                    recv_sem=recv_sems.at[c, 1, j],
                    device_id=(p,),
                    device_id_type=pl.DeviceIdType.MESH,
                )
                d.start()
                ds.append(d)
            return ds

        def phase2_wait(c):
            rows = CHUNKS[c][1]
            for j in range(3):
                pltpu.make_async_remote_copy(
                    src_ref=fin_ref.at[c, pl.ds(my, 1), pl.ds(0, rows)],
                    dst_ref=fin_ref.at[c, pl.ds(peers[j], 1), pl.ds(0, rows)],
                    send_sem=send_sems.at[c, 1, j],
                    recv_sem=recv_sems.at[c, 1, j],
                    device_id=(peers[j],),
                    device_id_type=pl.DeviceIdType.MESH,
                ).wait_recv()

        sends = []
        for h in range(HQ_SHARD):
            attn_head(0, h)
        finish_chunk(0)
        sends += phase1_start(0)
        for c in range(1, NC):
            for h in range(6):
                attn_head(c, h)
            phase1_reduce(c - 1)
            sends += phase2_start(c - 1)
            for h in range(6, HQ_SHARD):
                attn_head(c, h)
            finish_chunk(c)
            sends += phase1_start(c)
        phase1_reduce(NC - 1)
        sends += phase2_start(NC - 1)
        for c in range(NC):
            phase2_wait(c)
        for d in sends:
            d.wait_send()

        for c, (base, rows) in enumerate(CHUNKS):
            for qq in range(N_DEV):
                out_ref[0, base:base + rows,
                        qq * QC:(qq + 1) * QC] = fin_ref[c, qq, 0:rows]

    grid_spec = pltpu.PrefetchScalarGridSpec(
        num_scalar_prefetch=1,
        grid=(1,),
        in_specs=[
            pl.BlockSpec((1, SQ, D), lambda i, m: (0, 0, 0)),
            pl.BlockSpec((D, D), lambda i, m: (0, 0)),
            pl.BlockSpec((D, D), lambda i, m: (0, 0)),
            pl.BlockSpec(memory_space=pl.ANY),
            pl.BlockSpec(memory_space=pl.ANY),
        ],
        out_specs=pl.BlockSpec((1, SQ, D), lambda i, m: (0, 0, 0)),
        scratch_shapes=[
            pltpu.VMEM((KV_SHARD, SKV, DH), jnp.float32),
            pltpu.VMEM((KV_SHARD, SKV, DH), jnp.float32),
            pltpu.VMEM((SQ, D), jnp.float32),
            pltpu.VMEM((NC, N_DEV, MAXR, QC), jnp.float32),
            pltpu.VMEM((NC, 3, MAXR, QC), jnp.float32),
            pltpu.VMEM((NC, N_DEV, MAXR, QC), jnp.float32),
            pltpu.SemaphoreType.DMA((4,)),
            pltpu.SemaphoreType.DMA((NC, 2, 3)),
            pltpu.SemaphoreType.DMA((NC, 2, 3)),
        ],
    )
    return pl.pallas_call(
        body,
        grid_spec=grid_spec,
        out_shape=jax.ShapeDtypeStruct((1, SQ, D), jnp.float32),
        compiler_params=pltpu.CompilerParams(collective_id=0),
    )(my_pos.reshape(1), x, Wq, Wo, K_ext, V_ext)


# device time: 33663 ns/iter; 1.0533x vs baseline; 1.0323x over previous
import jax
import jax.numpy as jnp
from jax import lax
from jax.experimental import pallas as pl
from jax.experimental.pallas import tpu as pltpu

N_DEV = 4
SQ = 256
D = 1024
SKV = 4096
DH = 128
HQ_SHARD = 8
KV_SHARD = 2
QC = D // 4
SCALE = 0.08838834764831843

CHUNKS = ((0, 128), (128, 128))
NC = len(CHUNKS)
MAXR = max(r for _, r in CHUNKS)


def kernel(x, Wq, Wo, K_ext, V_ext):
    my_pos = lax.axis_index("i").astype(jnp.int32)

    def body(pos_ref, x_ref, wq_ref, wo_ref, k_any, v_any, out_ref,
             kv_ref, vv_ref, attn_ref, acc_ref, stage_ref, fin_ref,
             kv_sems, send_sems, recv_sems):
        my = pos_ref[0]
        peers = [jnp.bitwise_xor(my, j + 1) for j in range(3)]

        kv_copies = []
        for g in range(KV_SHARD):
            hd = 2 * my + g
            for j, (src, dst) in enumerate(((k_any, kv_ref), (v_any, vv_ref))):
                cp = pltpu.make_async_copy(
                    src.at[0, :, hd, :], dst.at[g], kv_sems.at[2 * g + j])
                cp.start()
                kv_copies.append(cp)

        barrier = pltpu.get_barrier_semaphore()
        for nbr in peers:
            pl.semaphore_signal(barrier, inc=1, device_id=(nbr,),
                                device_id_type=pl.DeviceIdType.MESH)
        pl.semaphore_wait(barrier, 3)

        q = jnp.dot(x_ref[0], wq_ref[:, :],
                    preferred_element_type=jnp.float32)
        qs = q * SCALE

        kv_waited = [False, False]

        def attn_head(c, h):
            g = h // 4
            if not kv_waited[g]:
                kv_copies[2 * g].wait()
                kv_copies[2 * g + 1].wait()
                kv_waited[g] = True
            base, rows = CHUNKS[c]
            q_h = qs[base:base + rows, h * DH:(h + 1) * DH]
            k_h = kv_ref[g]
            v_h = vv_ref[g]
            s = lax.dot_general(q_h, k_h, (((1,), (1,)), ((), ())),
                                preferred_element_type=jnp.float32)
            p = jnp.exp(s)
            l = jnp.sum(p, axis=1, keepdims=True)
            o = jnp.dot(p, v_h, preferred_element_type=jnp.float32) / l
            attn_ref[base:base + rows, h * DH:(h + 1) * DH] = o

        def finish_chunk(c):
            base, rows = CHUNKS[c]
            partial = jnp.dot(attn_ref[base:base + rows, :], wo_ref[:, :],
                              preferred_element_type=jnp.float32)
            for qq in range(N_DEV):
                acc_ref[c, qq, 0:rows] = partial[:, qq * QC:(qq + 1) * QC]

        def phase1_start(c):
            rows = CHUNKS[c][1]
            ds = []
            for j, p in enumerate(peers):
                d = pltpu.make_async_remote_copy(
                    src_ref=acc_ref.at[c, pl.ds(p, 1), pl.ds(0, rows)],
                    dst_ref=stage_ref.at[c, pl.ds(j, 1), pl.ds(0, rows)],
                    send_sem=send_sems.at[c, 0, j],
                    recv_sem=recv_sems.at[c, 0, j],
                    device_id=(p,),
                    device_id_type=pl.DeviceIdType.MESH,
                )
                d.start()
                ds.append(d)
            return ds

        def phase1_reduce(c):
            rows = CHUNKS[c][1]
            for j in range(3):
                pltpu.make_async_remote_copy(
                    src_ref=acc_ref.at[c, pl.ds(peers[j], 1), pl.ds(0, rows)],
                    dst_ref=stage_ref.at[c, pl.ds(j, 1), pl.ds(0, rows)],
                    send_sem=send_sems.at[c, 0, j],
                    recv_sem=recv_sems.at[c, 0, j],
                    device_id=(peers[j],),
                    device_id_type=pl.DeviceIdType.MESH,
                ).wait_recv()
            red = (acc_ref[c, pl.ds(my, 1), 0:rows]
                   + (stage_ref[c, 0:1, 0:rows] + stage_ref[c, 1:2, 0:rows])
                   + stage_ref[c, 2:3, 0:rows])
            fin_ref[c, pl.ds(my, 1), 0:rows] = red

        def phase2_start(c):
            rows = CHUNKS[c][1]
            ds = []
            for j, p in enumerate(peers):
                d = pltpu.make_async_remote_copy(
                    src_ref=fin_ref.at[c, pl.ds(my, 1), pl.ds(0, rows)],
                    dst_ref=fin_ref.at[c, pl.ds(my, 1), pl.ds(0, rows)],
                    send_sem=send_sems.at[c, 1, j],
                    recv_sem=recv_sems.at[c, 1, j],
                    device_id=(p,),
                    device_id_type=pl.DeviceIdType.MESH,
                )
                d.start()
                ds.append(d)
            return ds

        def phase2_wait(c):
            rows = CHUNKS[c][1]
            for j in range(3):
                pltpu.make_async_remote_copy(
                    src_ref=fin_ref.at[c, pl.ds(my, 1), pl.ds(0, rows)],
                    dst_ref=fin_ref.at[c, pl.ds(peers[j], 1), pl.ds(0, rows)],
                    send_sem=send_sems.at[c, 1, j],
                    recv_sem=recv_sems.at[c, 1, j],
                    device_id=(peers[j],),
                    device_id_type=pl.DeviceIdType.MESH,
                ).wait_recv()

        sends = []
        for h in range(HQ_SHARD):
            attn_head(0, h)
        finish_chunk(0)
        sends += phase1_start(0)
        for c in range(1, NC):
            for h in range(6):
                attn_head(c, h)
            phase1_reduce(c - 1)
            sends += phase2_start(c - 1)
            for h in range(6, HQ_SHARD):
                attn_head(c, h)
            finish_chunk(c)
            sends += phase1_start(c)
        phase1_reduce(NC - 1)
        sends += phase2_start(NC - 1)
        for c in range(NC):
            phase2_wait(c)
        for d in sends:
            d.wait_send()

        for c, (base, rows) in enumerate(CHUNKS):
            for qq in range(N_DEV):
                out_ref[0, base:base + rows,
                        qq * QC:(qq + 1) * QC] = fin_ref[c, qq, 0:rows]

    grid_spec = pltpu.PrefetchScalarGridSpec(
        num_scalar_prefetch=1,
        grid=(1,),
        in_specs=[
            pl.BlockSpec((1, SQ, D), lambda i, m: (0, 0, 0)),
            pl.BlockSpec((D, D), lambda i, m: (0, 0)),
            pl.BlockSpec((D, D), lambda i, m: (0, 0)),
            pl.BlockSpec(memory_space=pl.ANY),
            pl.BlockSpec(memory_space=pl.ANY),
        ],
        out_specs=pl.BlockSpec((1, SQ, D), lambda i, m: (0, 0, 0)),
        scratch_shapes=[
            pltpu.VMEM((KV_SHARD, SKV, DH), jnp.float32),
            pltpu.VMEM((KV_SHARD, SKV, DH), jnp.float32),
            pltpu.VMEM((SQ, D), jnp.float32),
            pltpu.VMEM((NC, N_DEV, MAXR, QC), jnp.float32),
            pltpu.VMEM((NC, 3, MAXR, QC), jnp.float32),
            pltpu.VMEM((NC, N_DEV, MAXR, QC), jnp.float32),
            pltpu.SemaphoreType.DMA((4,)),
            pltpu.SemaphoreType.DMA((NC, 2, 3)),
            pltpu.SemaphoreType.DMA((NC, 2, 3)),
        ],
    )
    return pl.pallas_call(
        body,
        grid_spec=grid_spec,
        out_shape=jax.ShapeDtypeStruct((1, SQ, D), jnp.float32),
        compiler_params=pltpu.CompilerParams(collective_id=0),
    )(my_pos.reshape(1), x, Wq, Wo, K_ext, V_ext)
